# baseline (speedup 1.0000x reference)
import functools

import jax
import jax.numpy as jnp
from jax import lax
from jax.experimental import pallas as pl
from jax.experimental.pallas import tpu as pltpu
from jax.experimental.pallas import tpu_sc as plsc

NC = 2
NS = 16
NW = NC * NS
CH = 80
LANES = 16


def _stage_node_tables(h, x_unused, AT, BT, be1):
    N, D = h.shape
    BN = 1000
    grid = N // BN

    def body(h_ref, at_ref, bt_ref, be1_ref, p_ref, q_ref):
        hb = h_ref[...]
        p_ref[...] = jnp.dot(hb, at_ref[...], preferred_element_type=jnp.float32)
        q_ref[...] = (jnp.dot(hb, bt_ref[...], preferred_element_type=jnp.float32)
                      + be1_ref[...])

    return pl.pallas_call(
        body,
        grid=(grid,),
        in_specs=[
            pl.BlockSpec((BN, D), lambda i: (i, 0)),
            pl.BlockSpec((D, D), lambda i: (0, 0)),
            pl.BlockSpec((D, D), lambda i: (0, 0)),
            pl.BlockSpec((1, D), lambda i: (0, 0)),
        ],
        out_specs=[
            pl.BlockSpec((BN, D), lambda i: (i, 0)),
            pl.BlockSpec((BN, D), lambda i: (i, 0)),
        ],
        out_shape=[jax.ShapeDtypeStruct((N, D), jnp.float32)] * 2,
    )(h, AT, BT, be1)


def _stage_gather(P, Q, xf, src, dst):
    E = src.shape[0]
    N, D = P.shape
    epw = E // NW
    iters = epw // CH
    groups = CH // LANES
    mesh = plsc.VectorSubcoreMesh(core_axis_name="c", subcore_axis_name="s")

    @functools.partial(
        pl.kernel,
        out_type=[
            jax.ShapeDtypeStruct((E, D), jnp.float32),
            jax.ShapeDtypeStruct((E,), jnp.float32),
            jax.ShapeDtypeStruct((E,), jnp.float32),
            jax.ShapeDtypeStruct((E,), jnp.float32),
        ],
        mesh=mesh,
        scratch_types=[
            pltpu.VMEM((3 * N,), jnp.float32),
            pltpu.VMEM((CH,), jnp.int32),
            pltpu.VMEM((CH,), jnp.int32),
            pltpu.VMEM((CH, D), jnp.float32),
            pltpu.VMEM((CH, D), jnp.float32),
            pltpu.VMEM((CH,), jnp.float32),
            pltpu.VMEM((CH,), jnp.float32),
            pltpu.VMEM((CH,), jnp.float32),
            pltpu.SemaphoreType.DMA,
            pltpu.SemaphoreType.DMA,
        ],
        compiler_params=pltpu.CompilerParams(needs_layout_passes=False),
    )
    def k(p_hbm, q_hbm, xf_hbm, src_hbm, dst_hbm, zr_hbm, rx_hbm, ry_hbm,
          rz_hbm, xtab, idxs, idxd, bufp, bufq, brx, bry, brz, semp, semq):
        wid = lax.axis_index("c") * NS + lax.axis_index("s")
        pltpu.sync_copy(xf_hbm, xtab)

        def body(i, carry):
            base = wid * epw + i * CH
            pltpu.sync_copy(src_hbm.at[pl.ds(base, CH)], idxs)
            pltpu.sync_copy(dst_hbm.at[pl.ds(base, CH)], idxd)
            cp = pltpu.async_copy(p_hbm.at[idxs], bufp, semp)
            cq = pltpu.async_copy(q_hbm.at[idxd], bufq, semq)

            for g in range(groups):
                sl = pl.ds(g * LANES, LANES)
                s3 = idxs[sl] * 3
                d3 = idxd[sl] * 3
                rx = (plsc.load_gather(xtab, [s3])
                      - plsc.load_gather(xtab, [d3]))
                ry = (plsc.load_gather(xtab, [s3 + 1])
                      - plsc.load_gather(xtab, [d3 + 1]))
                rz = (plsc.load_gather(xtab, [s3 + 2])
                      - plsc.load_gather(xtab, [d3 + 2]))
                brx[sl] = rx
                bry[sl] = ry
                brz[sl] = rz

            cp.wait()
            cq.wait()

            def row(r, carry2):
                for cc in range(D // LANES):
                    sl = pl.ds(cc * LANES, LANES)
                    plsc.addupdate(bufp.at[r, sl], bufq[r, sl])
                return carry2

            lax.fori_loop(0, CH, row, None)
            pltpu.sync_copy(bufp, zr_hbm.at[pl.ds(base, CH)])
            pltpu.sync_copy(brx, rx_hbm.at[pl.ds(base, CH)])
            pltpu.sync_copy(bry, ry_hbm.at[pl.ds(base, CH)])
            pltpu.sync_copy(brz, rz_hbm.at[pl.ds(base, CH)])
            return carry

        lax.fori_loop(0, iters, body, None)

    return k(P, Q, xf, src, dst)


def _stage_edge_mlp(zr, rx, ry, rz, ea, We2T, be2, Wc1T, bc1, Wc2T, c_row,
                    DmT):
    E, D = zr.shape
    BE = 512
    grid = E // BE

    def body(zr_ref, rx_ref, ry_ref, rz_ref, ea_ref, w2_ref, b2_ref,
             wc1_ref, bc1_ref, wc2_ref, cr_ref, dm_ref, m_ref, cx_ref,
             cy_ref, cz_ref):
        rxb = rx_ref[...]
        ryb = ry_ref[...]
        rzb = rz_ref[...]
        ds2 = (rxb * rxb + ryb * ryb + rzb * rzb)[:, None]
        z = (zr_ref[...] + ds2 * cr_ref[...]
             + jnp.dot(ea_ref[...], dm_ref[...],
                       preferred_element_type=jnp.float32))
        m1 = jax.nn.silu(z).astype(jnp.bfloat16)
        m = jax.nn.silu(
            jnp.dot(m1, w2_ref[...], preferred_element_type=jnp.float32)
            + b2_ref[...])
        cw = jax.nn.silu(
            jnp.dot(m.astype(jnp.bfloat16), wc1_ref[...],
                    preferred_element_type=jnp.float32)
            + bc1_ref[...])
        w0 = jnp.dot(cw.astype(jnp.bfloat16), wc2_ref[...],
                     preferred_element_type=jnp.float32)[:, 0]
        m_ref[...] = m
        cx_ref[...] = rxb * w0
        cy_ref[...] = ryb * w0
        cz_ref[...] = rzb * w0

    return pl.pallas_call(
        body,
        grid=(grid,),
        in_specs=[
            pl.BlockSpec((BE, D), lambda i: (i, 0)),
            pl.BlockSpec((BE,), lambda i: (i,)),
            pl.BlockSpec((BE,), lambda i: (i,)),
            pl.BlockSpec((BE,), lambda i: (i,)),
            pl.BlockSpec((BE, 4), lambda i: (i, 0)),
            pl.BlockSpec((D, D), lambda i: (0, 0)),
            pl.BlockSpec((1, D), lambda i: (0, 0)),
            pl.BlockSpec((D, D), lambda i: (0, 0)),
            pl.BlockSpec((1, D), lambda i: (0, 0)),
            pl.BlockSpec((D, 1), lambda i: (0, 0)),
            pl.BlockSpec((1, D), lambda i: (0, 0)),
            pl.BlockSpec((4, D), lambda i: (0, 0)),
        ],
        out_specs=[
            pl.BlockSpec((BE, D), lambda i: (i, 0)),
            pl.BlockSpec((BE,), lambda i: (i,)),
            pl.BlockSpec((BE,), lambda i: (i,)),
            pl.BlockSpec((BE,), lambda i: (i,)),
        ],
        out_shape=[
            jax.ShapeDtypeStruct((E, D), jnp.float32),
            jax.ShapeDtypeStruct((E,), jnp.float32),
            jax.ShapeDtypeStruct((E,), jnp.float32),
            jax.ShapeDtypeStruct((E,), jnp.float32),
        ],
    )(zr, rx, ry, rz, ea, We2T, be2, Wc1T, bc1, Wc2T, c_row, DmT)


def _stage_scatter(mc, cux, cuy, cuz, dst, N):
    E, D = mc.shape
    epw = E // NW
    iters = epw // CH
    groups = CH // LANES
    RPS = 640
    WB = 80
    full_chunks = RPS // WB
    last_chunks = (N - (NS - 1) * RPS) // WB
    mesh = plsc.VectorSubcoreMesh(core_axis_name="c", subcore_axis_name="s")

    @functools.partial(
        pl.kernel,
        out_type=[
            jax.ShapeDtypeStruct((NC, N, D), jnp.float32),
            jax.ShapeDtypeStruct((NW * N,), jnp.float32),
            jax.ShapeDtypeStruct((NW * N,), jnp.float32),
            jax.ShapeDtypeStruct((NW * N,), jnp.float32),
        ],
        mesh=mesh,
        scratch_types=[
            pltpu.VMEM((CH,), jnp.int32),
            pltpu.VMEM((CH, D), jnp.float32),
            pltpu.VMEM((N,), jnp.float32),
            pltpu.VMEM((N,), jnp.float32),
            pltpu.VMEM((N,), jnp.float32),
            pltpu.VMEM((CH,), jnp.float32),
            pltpu.VMEM((CH,), jnp.float32),
            pltpu.VMEM((CH,), jnp.float32),
            pltpu.VMEM_SHARED((N, D), jnp.float32),
        ],
        compiler_params=pltpu.CompilerParams(needs_layout_passes=False),
    )
    def k(mc_hbm, cux_hbm, cuy_hbm, cuz_hbm, dst_hbm, part_hbm, px_hbm,
          py_hbm, pz_hbm, idxd, mbuf, ax, ay, az, bcx, bcy, bcz, acc):
        c = lax.axis_index("c")
        s = lax.axis_index("s")
        wid = c * NS + s
        nchunks = jnp.where(s < NS - 1, full_chunks, last_chunks)
        zv = jnp.zeros((LANES,), jnp.float32)

        def zrow(r, carry):
            for cc in range(D // LANES):
                mbuf[r, pl.ds(cc * LANES, LANES)] = zv
            return carry

        lax.fori_loop(0, WB, zrow, None)

        def zcoord(r, carry):
            sl = pl.ds(r * LANES, LANES)
            ax[sl] = zv
            ay[sl] = zv
            az[sl] = zv
            return carry

        lax.fori_loop(0, N // LANES, zcoord, None)

        def zacc(j, carry):
            pltpu.sync_copy(mbuf, acc.at[pl.ds(s * RPS + j * WB, WB)])
            return carry

        lax.fori_loop(0, nchunks, zacc, None)
        plsc.subcore_barrier()

        def body(i, carry):
            base = wid * epw + i * CH
            pltpu.sync_copy(dst_hbm.at[pl.ds(base, CH)], idxd)
            pltpu.sync_copy(mc_hbm.at[pl.ds(base, CH)], mbuf)
            pltpu.sync_copy(mbuf, acc.at[idxd], add=True)
            pltpu.sync_copy(cux_hbm.at[pl.ds(base, CH)], bcx)
            pltpu.sync_copy(cuy_hbm.at[pl.ds(base, CH)], bcy)
            pltpu.sync_copy(cuz_hbm.at[pl.ds(base, CH)], bcz)
            for g in range(groups):
                sl = pl.ds(g * LANES, LANES)
                dv = idxd[sl]
                plsc.addupdate_scatter(ax, [dv], bcx[sl])
                plsc.addupdate_scatter(ay, [dv], bcy[sl])
                plsc.addupdate_scatter(az, [dv], bcz[sl])
            return carry

        lax.fori_loop(0, iters, body, None)
        plsc.subcore_barrier()

        def wb(j, carry):
            row0 = s * RPS + j * WB
            pltpu.sync_copy(acc.at[pl.ds(row0, WB)], mbuf)
            pltpu.sync_copy(mbuf, part_hbm.at[c, pl.ds(row0, WB)])
            return carry

        lax.fori_loop(0, nchunks, wb, None)
        pltpu.sync_copy(ax, px_hbm.at[pl.ds(wid * N, N)])
        pltpu.sync_copy(ay, py_hbm.at[pl.ds(wid * N, N)])
        pltpu.sync_copy(az, pz_hbm.at[pl.ds(wid * N, N)])

    part, px, py, pz = k(mc, cux, cuy, cuz, dst)
    BN = 1000
    def t(a):
        return a.reshape(NW, N // BN, BN).transpose(1, 0, 2)
    return part, t(px), t(py), t(pz)


def _stage_node_mlp(h, x, part, px, py, pz, W1hT, W1mT, bn1, Wn2T, bn2):
    N, D = h.shape
    BN = 1000
    grid = N // BN

    def body(h_ref, x_ref, part_ref, px_ref, py_ref, pz_ref, w1h_ref,
             w1m_ref, b1_ref, w2_ref, b2_ref, ho_ref, xo_ref):
        hb = h_ref[...]
        p = part_ref[...]
        m_i = p[0] + p[1]
        cx = jnp.sum(px_ref[0], axis=0)[:, None]
        cy = jnp.sum(py_ref[0], axis=0)[:, None]
        cz = jnp.sum(pz_ref[0], axis=0)[:, None]
        xo_ref[...] = x_ref[...] + jnp.concatenate([cx, cy, cz], axis=1)
        nh = jax.nn.silu(
            jnp.dot(hb, w1h_ref[...], preferred_element_type=jnp.float32)
            + jnp.dot(m_i, w1m_ref[...], preferred_element_type=jnp.float32)
            + b1_ref[...])
        nh = (jnp.dot(nh, w2_ref[...], preferred_element_type=jnp.float32)
              + b2_ref[...])
        ho_ref[...] = hb + nh

    return pl.pallas_call(
        body,
        grid=(grid,),
        in_specs=[
            pl.BlockSpec((BN, D), lambda i: (i, 0)),
            pl.BlockSpec((BN, 3), lambda i: (i, 0)),
            pl.BlockSpec((NC, BN, D), lambda i: (0, i, 0)),
            pl.BlockSpec((1, NW, BN), lambda i: (i, 0, 0)),
            pl.BlockSpec((1, NW, BN), lambda i: (i, 0, 0)),
            pl.BlockSpec((1, NW, BN), lambda i: (i, 0, 0)),
            pl.BlockSpec((D, D), lambda i: (0, 0)),
            pl.BlockSpec((D, D), lambda i: (0, 0)),
            pl.BlockSpec((1, D), lambda i: (0, 0)),
            pl.BlockSpec((D, D), lambda i: (0, 0)),
            pl.BlockSpec((1, D), lambda i: (0, 0)),
        ],
        out_specs=[
            pl.BlockSpec((BN, D), lambda i: (i, 0)),
            pl.BlockSpec((BN, 3), lambda i: (i, 0)),
        ],
        out_shape=[
            jax.ShapeDtypeStruct((N, D), jnp.float32),
            jax.ShapeDtypeStruct((N, 3), jnp.float32),
        ],
    )(h, x, part, px, py, pz, W1hT, W1mT, bn1, Wn2T, bn2)


def kernel(h, x, edge_index, edge_attr, We1, be1, We2, be2, Wn1, bn1, Wn2,
           bn2, Wc1, bc1, Wc2):
    N, D = h.shape
    src = edge_index[0]
    dst = edge_index[1]
    AT = We1[:, :D].T
    BT = We1[:, D:2 * D].T
    c_row = We1[:, 2 * D].reshape(1, D)
    DmT = We1[:, 2 * D + 1:].T
    xf = x.reshape(-1)
    P, Q = _stage_node_tables(h, x, AT, BT, be1.reshape(1, D))
    zr, rx, ry, rz = _stage_gather(P, Q, xf, src, dst)
    mc, cux, cuy, cuz = _stage_edge_mlp(
        zr, rx, ry, rz, edge_attr, We2.T.astype(jnp.bfloat16),
        be2.reshape(1, D), Wc1.T.astype(jnp.bfloat16), bc1.reshape(1, D),
        Wc2.T.astype(jnp.bfloat16), c_row, DmT)
    part, px, py, pz = _stage_scatter(mc, cux, cuy, cuz, dst, N)
    h_out, x_out = _stage_node_mlp(h, x, part, px, py, pz, Wn1[:, :D].T,
                                   Wn1[:, D:].T, bn1.reshape(1, D), Wn2.T,
                                   bn2.reshape(1, D))
    return (h_out, x_out)


# speedup vs baseline: 1.0072x; 1.0072x over previous
import functools

import jax
import jax.numpy as jnp
from jax import lax
from jax.experimental import pallas as pl
from jax.experimental.pallas import tpu as pltpu
from jax.experimental.pallas import tpu_sc as plsc

NC = 2
NS = 16
NW = NC * NS
CH = 80
LANES = 16


def _stage_node_tables(h, x_unused, AT, BT, be1):
    N, D = h.shape
    BN = 1000
    grid = N // BN

    def body(h_ref, at_ref, bt_ref, be1_ref, p_ref, q_ref):
        hb = h_ref[...]
        p_ref[...] = jnp.dot(hb, at_ref[...], preferred_element_type=jnp.float32)
        q_ref[...] = (jnp.dot(hb, bt_ref[...], preferred_element_type=jnp.float32)
                      + be1_ref[...])

    return pl.pallas_call(
        body,
        grid=(grid,),
        in_specs=[
            pl.BlockSpec((BN, D), lambda i: (i, 0)),
            pl.BlockSpec((D, D), lambda i: (0, 0)),
            pl.BlockSpec((D, D), lambda i: (0, 0)),
            pl.BlockSpec((1, D), lambda i: (0, 0)),
        ],
        out_specs=[
            pl.BlockSpec((BN, D), lambda i: (i, 0)),
            pl.BlockSpec((BN, D), lambda i: (i, 0)),
        ],
        out_shape=[jax.ShapeDtypeStruct((N, D), jnp.float32)] * 2,
    )(h, AT, BT, be1)


def _stage_gather(P, Q, xf, src, dst):
    E = src.shape[0]
    N, D = P.shape
    epw = E // NW
    iters = epw // CH
    groups = CH // LANES
    mesh = plsc.VectorSubcoreMesh(core_axis_name="c", subcore_axis_name="s")

    @functools.partial(
        pl.kernel,
        out_type=[
            jax.ShapeDtypeStruct((E, D), jnp.float32),
            jax.ShapeDtypeStruct((E,), jnp.float32),
            jax.ShapeDtypeStruct((E,), jnp.float32),
            jax.ShapeDtypeStruct((E,), jnp.float32),
        ],
        mesh=mesh,
        scratch_types=[
            pltpu.VMEM((3 * N,), jnp.float32),
            pltpu.VMEM((CH,), jnp.int32),
            pltpu.VMEM((CH,), jnp.int32),
            pltpu.VMEM((CH, D), jnp.float32),
            pltpu.VMEM((CH, D), jnp.float32),
            pltpu.VMEM((CH,), jnp.float32),
            pltpu.VMEM((CH,), jnp.float32),
            pltpu.VMEM((CH,), jnp.float32),
            pltpu.SemaphoreType.DMA,
            pltpu.SemaphoreType.DMA,
        ],
        compiler_params=pltpu.CompilerParams(needs_layout_passes=False),
    )
    def k(p_hbm, q_hbm, xf_hbm, src_hbm, dst_hbm, zr_hbm, rx_hbm, ry_hbm,
          rz_hbm, xtab, idxs, idxd, bufp, bufq, brx, bry, brz, semp, semq):
        wid = lax.axis_index("c") * NS + lax.axis_index("s")
        pltpu.sync_copy(xf_hbm, xtab)

        def body(i, carry):
            base = wid * epw + i * CH
            pltpu.sync_copy(src_hbm.at[pl.ds(base, CH)], idxs)
            pltpu.sync_copy(dst_hbm.at[pl.ds(base, CH)], idxd)
            cp = pltpu.async_copy(p_hbm.at[idxs], bufp, semp)
            cq = pltpu.async_copy(q_hbm.at[idxd], bufq, semq)

            for g in range(groups):
                sl = pl.ds(g * LANES, LANES)
                s3 = idxs[sl] * 3
                d3 = idxd[sl] * 3
                rx = (plsc.load_gather(xtab, [s3])
                      - plsc.load_gather(xtab, [d3]))
                ry = (plsc.load_gather(xtab, [s3 + 1])
                      - plsc.load_gather(xtab, [d3 + 1]))
                rz = (plsc.load_gather(xtab, [s3 + 2])
                      - plsc.load_gather(xtab, [d3 + 2]))
                brx[sl] = rx
                bry[sl] = ry
                brz[sl] = rz

            cp.wait()
            cq.wait()

            def row(r, carry2):
                for cc in range(D // LANES):
                    sl = pl.ds(cc * LANES, LANES)
                    plsc.addupdate(bufp.at[r, sl], bufq[r, sl])
                return carry2

            lax.fori_loop(0, CH, row, None)
            pltpu.sync_copy(bufp, zr_hbm.at[pl.ds(base, CH)])
            pltpu.sync_copy(brx, rx_hbm.at[pl.ds(base, CH)])
            pltpu.sync_copy(bry, ry_hbm.at[pl.ds(base, CH)])
            pltpu.sync_copy(brz, rz_hbm.at[pl.ds(base, CH)])
            return carry

        lax.fori_loop(0, iters, body, None)

    return k(P, Q, xf, src, dst)


def _stage_edge_mlp(zr, rx, ry, rz, ea, We2T, be2, Wc1T, bc1, Wc2T, c_row,
                    DmT):
    E, D = zr.shape
    BE = 512
    grid = E // BE

    def body(zr_ref, rx_ref, ry_ref, rz_ref, ea_ref, w2_ref, b2_ref,
             wc1_ref, bc1_ref, wc2_ref, cr_ref, dm_ref, m_ref, cx_ref,
             cy_ref, cz_ref):
        rxb = rx_ref[...]
        ryb = ry_ref[...]
        rzb = rz_ref[...]
        ds2 = (rxb * rxb + ryb * ryb + rzb * rzb)[:, None]
        z = (zr_ref[...] + ds2 * cr_ref[...]
             + jnp.dot(ea_ref[...], dm_ref[...],
                       preferred_element_type=jnp.float32))
        m1 = jax.nn.silu(z).astype(jnp.bfloat16)
        m = jax.nn.silu(
            jnp.dot(m1, w2_ref[...], preferred_element_type=jnp.float32)
            + b2_ref[...])
        cw = jax.nn.silu(
            jnp.dot(m.astype(jnp.bfloat16), wc1_ref[...],
                    preferred_element_type=jnp.float32)
            + bc1_ref[...])
        w0 = jnp.sum(cw * wc2_ref[...], axis=1)
        m_ref[...] = m
        cx_ref[...] = rxb * w0
        cy_ref[...] = ryb * w0
        cz_ref[...] = rzb * w0

    return pl.pallas_call(
        body,
        grid=(grid,),
        in_specs=[
            pl.BlockSpec((BE, D), lambda i: (i, 0)),
            pl.BlockSpec((BE,), lambda i: (i,)),
            pl.BlockSpec((BE,), lambda i: (i,)),
            pl.BlockSpec((BE,), lambda i: (i,)),
            pl.BlockSpec((BE, 4), lambda i: (i, 0)),
            pl.BlockSpec((D, D), lambda i: (0, 0)),
            pl.BlockSpec((1, D), lambda i: (0, 0)),
            pl.BlockSpec((D, D), lambda i: (0, 0)),
            pl.BlockSpec((1, D), lambda i: (0, 0)),
            pl.BlockSpec((1, D), lambda i: (0, 0)),
            pl.BlockSpec((1, D), lambda i: (0, 0)),
            pl.BlockSpec((4, D), lambda i: (0, 0)),
        ],
        out_specs=[
            pl.BlockSpec((BE, D), lambda i: (i, 0)),
            pl.BlockSpec((BE,), lambda i: (i,)),
            pl.BlockSpec((BE,), lambda i: (i,)),
            pl.BlockSpec((BE,), lambda i: (i,)),
        ],
        out_shape=[
            jax.ShapeDtypeStruct((E, D), jnp.float32),
            jax.ShapeDtypeStruct((E,), jnp.float32),
            jax.ShapeDtypeStruct((E,), jnp.float32),
            jax.ShapeDtypeStruct((E,), jnp.float32),
        ],
    )(zr, rx, ry, rz, ea, We2T, be2, Wc1T, bc1, Wc2T, c_row, DmT)


def _stage_scatter(mc, cux, cuy, cuz, dst, N):
    E, D = mc.shape
    epw = E // NW
    iters = epw // CH
    groups = CH // LANES
    RPS = 640
    WB = 80
    full_chunks = RPS // WB
    last_chunks = (N - (NS - 1) * RPS) // WB
    mesh = plsc.VectorSubcoreMesh(core_axis_name="c", subcore_axis_name="s")

    @functools.partial(
        pl.kernel,
        out_type=[
            jax.ShapeDtypeStruct((NC, N, D), jnp.float32),
            jax.ShapeDtypeStruct((NW * N,), jnp.float32),
            jax.ShapeDtypeStruct((NW * N,), jnp.float32),
            jax.ShapeDtypeStruct((NW * N,), jnp.float32),
        ],
        mesh=mesh,
        scratch_types=[
            pltpu.VMEM((CH,), jnp.int32),
            pltpu.VMEM((CH, D), jnp.float32),
            pltpu.VMEM((N,), jnp.float32),
            pltpu.VMEM((N,), jnp.float32),
            pltpu.VMEM((N,), jnp.float32),
            pltpu.VMEM((CH,), jnp.float32),
            pltpu.VMEM((CH,), jnp.float32),
            pltpu.VMEM((CH,), jnp.float32),
            pltpu.VMEM_SHARED((N, D), jnp.float32),
        ],
        compiler_params=pltpu.CompilerParams(needs_layout_passes=False),
    )
    def k(mc_hbm, cux_hbm, cuy_hbm, cuz_hbm, dst_hbm, part_hbm, px_hbm,
          py_hbm, pz_hbm, idxd, mbuf, ax, ay, az, bcx, bcy, bcz, acc):
        c = lax.axis_index("c")
        s = lax.axis_index("s")
        wid = c * NS + s
        nchunks = jnp.where(s < NS - 1, full_chunks, last_chunks)
        zv = jnp.zeros((LANES,), jnp.float32)

        def zrow(r, carry):
            for cc in range(D // LANES):
                mbuf[r, pl.ds(cc * LANES, LANES)] = zv
            return carry

        lax.fori_loop(0, WB, zrow, None)

        def zcoord(r, carry):
            sl = pl.ds(r * LANES, LANES)
            ax[sl] = zv
            ay[sl] = zv
            az[sl] = zv
            return carry

        lax.fori_loop(0, N // LANES, zcoord, None)

        def zacc(j, carry):
            pltpu.sync_copy(mbuf, acc.at[pl.ds(s * RPS + j * WB, WB)])
            return carry

        lax.fori_loop(0, nchunks, zacc, None)
        plsc.subcore_barrier()

        def body(i, carry):
            base = wid * epw + i * CH
            pltpu.sync_copy(dst_hbm.at[pl.ds(base, CH)], idxd)
            pltpu.sync_copy(mc_hbm.at[pl.ds(base, CH)], mbuf)
            pltpu.sync_copy(mbuf, acc.at[idxd], add=True)
            pltpu.sync_copy(cux_hbm.at[pl.ds(base, CH)], bcx)
            pltpu.sync_copy(cuy_hbm.at[pl.ds(base, CH)], bcy)
            pltpu.sync_copy(cuz_hbm.at[pl.ds(base, CH)], bcz)
            for g in range(groups):
                sl = pl.ds(g * LANES, LANES)
                dv = idxd[sl]
                plsc.addupdate_scatter(ax, [dv], bcx[sl])
                plsc.addupdate_scatter(ay, [dv], bcy[sl])
                plsc.addupdate_scatter(az, [dv], bcz[sl])
            return carry

        lax.fori_loop(0, iters, body, None)
        plsc.subcore_barrier()

        def wb(j, carry):
            row0 = s * RPS + j * WB
            pltpu.sync_copy(acc.at[pl.ds(row0, WB)], mbuf)
            pltpu.sync_copy(mbuf, part_hbm.at[c, pl.ds(row0, WB)])
            return carry

        lax.fori_loop(0, nchunks, wb, None)
        pltpu.sync_copy(ax, px_hbm.at[pl.ds(wid * N, N)])
        pltpu.sync_copy(ay, py_hbm.at[pl.ds(wid * N, N)])
        pltpu.sync_copy(az, pz_hbm.at[pl.ds(wid * N, N)])

    part, px, py, pz = k(mc, cux, cuy, cuz, dst)
    BN = 1000
    def t(a):
        return a.reshape(NW, N // BN, BN).transpose(1, 0, 2)
    return part, t(px), t(py), t(pz)


def _stage_node_mlp(h, x, part, px, py, pz, W1hT, W1mT, bn1, Wn2T, bn2):
    N, D = h.shape
    BN = 1000
    grid = N // BN

    def body(h_ref, x_ref, part_ref, px_ref, py_ref, pz_ref, w1h_ref,
             w1m_ref, b1_ref, w2_ref, b2_ref, ho_ref, xo_ref):
        hb = h_ref[...]
        p = part_ref[...]
        m_i = p[0] + p[1]
        cx = jnp.sum(px_ref[0], axis=0)[:, None]
        cy = jnp.sum(py_ref[0], axis=0)[:, None]
        cz = jnp.sum(pz_ref[0], axis=0)[:, None]
        xo_ref[...] = x_ref[...] + jnp.concatenate([cx, cy, cz], axis=1)
        nh = jax.nn.silu(
            jnp.dot(hb, w1h_ref[...], preferred_element_type=jnp.float32)
            + jnp.dot(m_i, w1m_ref[...], preferred_element_type=jnp.float32)
            + b1_ref[...])
        nh = (jnp.dot(nh, w2_ref[...], preferred_element_type=jnp.float32)
              + b2_ref[...])
        ho_ref[...] = hb + nh

    return pl.pallas_call(
        body,
        grid=(grid,),
        in_specs=[
            pl.BlockSpec((BN, D), lambda i: (i, 0)),
            pl.BlockSpec((BN, 3), lambda i: (i, 0)),
            pl.BlockSpec((NC, BN, D), lambda i: (0, i, 0)),
            pl.BlockSpec((1, NW, BN), lambda i: (i, 0, 0)),
            pl.BlockSpec((1, NW, BN), lambda i: (i, 0, 0)),
            pl.BlockSpec((1, NW, BN), lambda i: (i, 0, 0)),
            pl.BlockSpec((D, D), lambda i: (0, 0)),
            pl.BlockSpec((D, D), lambda i: (0, 0)),
            pl.BlockSpec((1, D), lambda i: (0, 0)),
            pl.BlockSpec((D, D), lambda i: (0, 0)),
            pl.BlockSpec((1, D), lambda i: (0, 0)),
        ],
        out_specs=[
            pl.BlockSpec((BN, D), lambda i: (i, 0)),
            pl.BlockSpec((BN, 3), lambda i: (i, 0)),
        ],
        out_shape=[
            jax.ShapeDtypeStruct((N, D), jnp.float32),
            jax.ShapeDtypeStruct((N, 3), jnp.float32),
        ],
    )(h, x, part, px, py, pz, W1hT, W1mT, bn1, Wn2T, bn2)


def kernel(h, x, edge_index, edge_attr, We1, be1, We2, be2, Wn1, bn1, Wn2,
           bn2, Wc1, bc1, Wc2):
    N, D = h.shape
    src = edge_index[0]
    dst = edge_index[1]
    AT = We1[:, :D].T
    BT = We1[:, D:2 * D].T
    c_row = We1[:, 2 * D].reshape(1, D)
    DmT = We1[:, 2 * D + 1:].T
    xf = x.reshape(-1)
    P, Q = _stage_node_tables(h, x, AT, BT, be1.reshape(1, D))
    zr, rx, ry, rz = _stage_gather(P, Q, xf, src, dst)
    mc, cux, cuy, cuz = _stage_edge_mlp(
        zr, rx, ry, rz, edge_attr, We2.T.astype(jnp.bfloat16),
        be2.reshape(1, D), Wc1.T.astype(jnp.bfloat16), bc1.reshape(1, D),
        Wc2.reshape(1, D), c_row, DmT)
    part, px, py, pz = _stage_scatter(mc, cux, cuy, cuz, dst, N)
    h_out, x_out = _stage_node_mlp(h, x, part, px, py, pz, Wn1[:, :D].T,
                                   Wn1[:, D:].T, bn1.reshape(1, D), Wn2.T,
                                   bn2.reshape(1, D))
    return (h_out, x_out)


# speedup vs baseline: 1.1909x; 1.1824x over previous
import functools

import jax
import jax.numpy as jnp
from jax import lax
from jax.experimental import pallas as pl
from jax.experimental.pallas import tpu as pltpu
from jax.experimental.pallas import tpu_sc as plsc

NC = 2
NS = 16
NW = NC * NS
CH = 80
LANES = 16


def _stage_node_tables(h, x_unused, AT, BT, be1):
    N, D = h.shape
    BN = 1000
    grid = N // BN

    def body(h_ref, at_ref, bt_ref, be1_ref, p_ref, q_ref):
        hb = h_ref[...]
        p_ref[...] = jnp.dot(hb, at_ref[...], preferred_element_type=jnp.float32)
        q_ref[...] = (jnp.dot(hb, bt_ref[...], preferred_element_type=jnp.float32)
                      + be1_ref[...])

    return pl.pallas_call(
        body,
        grid=(grid,),
        in_specs=[
            pl.BlockSpec((BN, D), lambda i: (i, 0)),
            pl.BlockSpec((D, D), lambda i: (0, 0)),
            pl.BlockSpec((D, D), lambda i: (0, 0)),
            pl.BlockSpec((1, D), lambda i: (0, 0)),
        ],
        out_specs=[
            pl.BlockSpec((BN, D), lambda i: (i, 0)),
            pl.BlockSpec((BN, D), lambda i: (i, 0)),
        ],
        out_shape=[jax.ShapeDtypeStruct((N, D), jnp.float32)] * 2,
    )(h, AT, BT, be1)


def _stage_gather(P, Q, xf, src, dst):
    E = src.shape[0]
    N, D = P.shape
    epw = E // NW
    iters = epw // CH
    groups = CH // LANES
    mesh = plsc.VectorSubcoreMesh(core_axis_name="c", subcore_axis_name="s")

    @functools.partial(
        pl.kernel,
        out_type=[
            jax.ShapeDtypeStruct((E, D), jnp.float32),
            jax.ShapeDtypeStruct((E,), jnp.float32),
            jax.ShapeDtypeStruct((E,), jnp.float32),
            jax.ShapeDtypeStruct((E,), jnp.float32),
        ],
        mesh=mesh,
        scratch_types=[
            pltpu.VMEM((3 * N,), jnp.float32),
            pltpu.VMEM((epw,), jnp.int32),
            pltpu.VMEM((epw,), jnp.int32),
            pltpu.VMEM((CH, D), jnp.float32),
            pltpu.VMEM((CH, D), jnp.float32),
            pltpu.VMEM((CH, D), jnp.float32),
            pltpu.VMEM((CH, D), jnp.float32),
            pltpu.VMEM((epw,), jnp.float32),
            pltpu.VMEM((epw,), jnp.float32),
            pltpu.VMEM((epw,), jnp.float32),
            pltpu.SemaphoreType.DMA,
            pltpu.SemaphoreType.DMA,
            pltpu.SemaphoreType.DMA,
            pltpu.SemaphoreType.DMA,
            pltpu.SemaphoreType.DMA,
            pltpu.SemaphoreType.DMA,
        ],
        compiler_params=pltpu.CompilerParams(needs_layout_passes=False),
    )
    def k(p_hbm, q_hbm, xf_hbm, src_hbm, dst_hbm, zr_hbm, rx_hbm, ry_hbm,
          rz_hbm, xtab, idxs, idxd, bufp0, bufq0, bufp1, bufq1, brx, bry,
          brz, semp0, semq0, semp1, semq1, semo0, semo1):
        wid = lax.axis_index("c") * NS + lax.axis_index("s")
        ebase = wid * epw
        pltpu.sync_copy(xf_hbm, xtab)
        pltpu.sync_copy(src_hbm.at[pl.ds(ebase, epw)], idxs)
        pltpu.sync_copy(dst_hbm.at[pl.ds(ebase, epw)], idxd)
        bufp = (bufp0, bufp1)
        bufq = (bufq0, bufq1)
        semp = (semp0, semp1)
        semq = (semq0, semq1)
        semo = (semo0, semo1)

        def fire(kk, st):
            isl = pl.ds(kk * CH, CH)
            pltpu.async_copy(p_hbm.at[idxs.at[isl]], bufp[st], semp[st])
            pltpu.async_copy(q_hbm.at[idxd.at[isl]], bufq[st], semq[st])

        def process(kk, st):
            for g in range(groups):
                sl = pl.ds(kk * CH + g * LANES, LANES)
                s3 = idxs[sl] * 3
                d3 = idxd[sl] * 3
                brx[sl] = (plsc.load_gather(xtab, [s3])
                           - plsc.load_gather(xtab, [d3]))
                bry[sl] = (plsc.load_gather(xtab, [s3 + 1])
                           - plsc.load_gather(xtab, [d3 + 1]))
                brz[sl] = (plsc.load_gather(xtab, [s3 + 2])
                           - plsc.load_gather(xtab, [d3 + 2]))
            pltpu.make_async_copy(p_hbm.at[pl.ds(0, CH)], bufp[st],
                                  semp[st]).wait()
            pltpu.make_async_copy(q_hbm.at[pl.ds(0, CH)], bufq[st],
                                  semq[st]).wait()

            def row(r, carry2):
                for cc in range(D // LANES):
                    sl = pl.ds(cc * LANES, LANES)
                    plsc.addupdate(bufp[st].at[r, sl], bufq[st][r, sl])
                return carry2

            lax.fori_loop(0, CH, row, None)
            pltpu.async_copy(bufp[st], zr_hbm.at[pl.ds(ebase + kk * CH, CH)],
                             semo[st])

        def wait_out(st):
            pltpu.make_async_copy(bufp[st], zr_hbm.at[pl.ds(0, CH)],
                                  semo[st]).wait()

        fire(0, 0)

        def body(j, carry):
            ka = 2 * j + 1

            @pl.when(j > 0)
            def _():
                wait_out(1)

            fire(ka, 1)
            process(ka - 1, 0)
            wait_out(0)
            fire(ka + 1, 0)
            process(ka, 1)
            return carry

        lax.fori_loop(0, (iters - 1) // 2, body, None)
        wait_out(1)
        process(iters - 1, 0)
        wait_out(0)
        pltpu.sync_copy(brx, rx_hbm.at[pl.ds(ebase, epw)])
        pltpu.sync_copy(bry, ry_hbm.at[pl.ds(ebase, epw)])
        pltpu.sync_copy(brz, rz_hbm.at[pl.ds(ebase, epw)])

    return k(P, Q, xf, src, dst)


def _stage_edge_mlp(zr, rx, ry, rz, ea, We2T, be2, Wc1T, bc1, Wc2T, c_row,
                    DmT):
    E, D = zr.shape
    BE = 512
    grid = E // BE

    def body(zr_ref, rx_ref, ry_ref, rz_ref, ea_ref, w2_ref, b2_ref,
             wc1_ref, bc1_ref, wc2_ref, cr_ref, dm_ref, m_ref, cx_ref,
             cy_ref, cz_ref):
        rxb = rx_ref[...]
        ryb = ry_ref[...]
        rzb = rz_ref[...]
        ds2 = (rxb * rxb + ryb * ryb + rzb * rzb)[:, None]
        z = (zr_ref[...] + ds2 * cr_ref[...]
             + jnp.dot(ea_ref[...], dm_ref[...],
                       preferred_element_type=jnp.float32))
        m1 = jax.nn.silu(z).astype(jnp.bfloat16)
        m = jax.nn.silu(
            jnp.dot(m1, w2_ref[...], preferred_element_type=jnp.float32)
            + b2_ref[...])
        cw = jax.nn.silu(
            jnp.dot(m.astype(jnp.bfloat16), wc1_ref[...],
                    preferred_element_type=jnp.float32)
            + bc1_ref[...])
        w0 = jnp.sum(cw * wc2_ref[...], axis=1)
        m_ref[...] = m
        cx_ref[...] = rxb * w0
        cy_ref[...] = ryb * w0
        cz_ref[...] = rzb * w0

    return pl.pallas_call(
        body,
        grid=(grid,),
        in_specs=[
            pl.BlockSpec((BE, D), lambda i: (i, 0)),
            pl.BlockSpec((BE,), lambda i: (i,)),
            pl.BlockSpec((BE,), lambda i: (i,)),
            pl.BlockSpec((BE,), lambda i: (i,)),
            pl.BlockSpec((BE, 4), lambda i: (i, 0)),
            pl.BlockSpec((D, D), lambda i: (0, 0)),
            pl.BlockSpec((1, D), lambda i: (0, 0)),
            pl.BlockSpec((D, D), lambda i: (0, 0)),
            pl.BlockSpec((1, D), lambda i: (0, 0)),
            pl.BlockSpec((1, D), lambda i: (0, 0)),
            pl.BlockSpec((1, D), lambda i: (0, 0)),
            pl.BlockSpec((4, D), lambda i: (0, 0)),
        ],
        out_specs=[
            pl.BlockSpec((BE, D), lambda i: (i, 0)),
            pl.BlockSpec((BE,), lambda i: (i,)),
            pl.BlockSpec((BE,), lambda i: (i,)),
            pl.BlockSpec((BE,), lambda i: (i,)),
        ],
        out_shape=[
            jax.ShapeDtypeStruct((E, D), jnp.float32),
            jax.ShapeDtypeStruct((E,), jnp.float32),
            jax.ShapeDtypeStruct((E,), jnp.float32),
            jax.ShapeDtypeStruct((E,), jnp.float32),
        ],
    )(zr, rx, ry, rz, ea, We2T, be2, Wc1T, bc1, Wc2T, c_row, DmT)


def _stage_scatter(mc, cux, cuy, cuz, dst, N):
    E, D = mc.shape
    epw = E // NW
    iters = epw // CH
    groups = CH // LANES
    RPS = 640
    WB = 80
    full_chunks = RPS // WB
    last_chunks = (N - (NS - 1) * RPS) // WB
    mesh = plsc.VectorSubcoreMesh(core_axis_name="c", subcore_axis_name="s")

    @functools.partial(
        pl.kernel,
        out_type=[
            jax.ShapeDtypeStruct((NC, N, D), jnp.float32),
            jax.ShapeDtypeStruct((NW * N,), jnp.float32),
            jax.ShapeDtypeStruct((NW * N,), jnp.float32),
            jax.ShapeDtypeStruct((NW * N,), jnp.float32),
        ],
        mesh=mesh,
        scratch_types=[
            pltpu.VMEM((CH,), jnp.int32),
            pltpu.VMEM((CH, D), jnp.float32),
            pltpu.VMEM((N,), jnp.float32),
            pltpu.VMEM((N,), jnp.float32),
            pltpu.VMEM((N,), jnp.float32),
            pltpu.VMEM((CH,), jnp.float32),
            pltpu.VMEM((CH,), jnp.float32),
            pltpu.VMEM((CH,), jnp.float32),
            pltpu.VMEM_SHARED((N, D), jnp.float32),
        ],
        compiler_params=pltpu.CompilerParams(needs_layout_passes=False),
    )
    def k(mc_hbm, cux_hbm, cuy_hbm, cuz_hbm, dst_hbm, part_hbm, px_hbm,
          py_hbm, pz_hbm, idxd, mbuf, ax, ay, az, bcx, bcy, bcz, acc):
        c = lax.axis_index("c")
        s = lax.axis_index("s")
        wid = c * NS + s
        nchunks = jnp.where(s < NS - 1, full_chunks, last_chunks)
        zv = jnp.zeros((LANES,), jnp.float32)

        def zrow(r, carry):
            for cc in range(D // LANES):
                mbuf[r, pl.ds(cc * LANES, LANES)] = zv
            return carry

        lax.fori_loop(0, WB, zrow, None)

        def zcoord(r, carry):
            sl = pl.ds(r * LANES, LANES)
            ax[sl] = zv
            ay[sl] = zv
            az[sl] = zv
            return carry

        lax.fori_loop(0, N // LANES, zcoord, None)

        def zacc(j, carry):
            pltpu.sync_copy(mbuf, acc.at[pl.ds(s * RPS + j * WB, WB)])
            return carry

        lax.fori_loop(0, nchunks, zacc, None)
        plsc.subcore_barrier()

        def body(i, carry):
            base = wid * epw + i * CH
            pltpu.sync_copy(dst_hbm.at[pl.ds(base, CH)], idxd)
            pltpu.sync_copy(mc_hbm.at[pl.ds(base, CH)], mbuf)
            pltpu.sync_copy(mbuf, acc.at[idxd], add=True)
            pltpu.sync_copy(cux_hbm.at[pl.ds(base, CH)], bcx)
            pltpu.sync_copy(cuy_hbm.at[pl.ds(base, CH)], bcy)
            pltpu.sync_copy(cuz_hbm.at[pl.ds(base, CH)], bcz)
            for g in range(groups):
                sl = pl.ds(g * LANES, LANES)
                dv = idxd[sl]
                plsc.addupdate_scatter(ax, [dv], bcx[sl])
                plsc.addupdate_scatter(ay, [dv], bcy[sl])
                plsc.addupdate_scatter(az, [dv], bcz[sl])
            return carry

        lax.fori_loop(0, iters, body, None)
        plsc.subcore_barrier()

        def wb(j, carry):
            row0 = s * RPS + j * WB
            pltpu.sync_copy(acc.at[pl.ds(row0, WB)], mbuf)
            pltpu.sync_copy(mbuf, part_hbm.at[c, pl.ds(row0, WB)])
            return carry

        lax.fori_loop(0, nchunks, wb, None)
        pltpu.sync_copy(ax, px_hbm.at[pl.ds(wid * N, N)])
        pltpu.sync_copy(ay, py_hbm.at[pl.ds(wid * N, N)])
        pltpu.sync_copy(az, pz_hbm.at[pl.ds(wid * N, N)])

    part, px, py, pz = k(mc, cux, cuy, cuz, dst)
    BN = 1000
    def t(a):
        return a.reshape(NW, N // BN, BN).transpose(1, 0, 2)
    return part, t(px), t(py), t(pz)


def _stage_node_mlp(h, x, part, px, py, pz, W1hT, W1mT, bn1, Wn2T, bn2):
    N, D = h.shape
    BN = 1000
    grid = N // BN

    def body(h_ref, x_ref, part_ref, px_ref, py_ref, pz_ref, w1h_ref,
             w1m_ref, b1_ref, w2_ref, b2_ref, ho_ref, xo_ref):
        hb = h_ref[...]
        p = part_ref[...]
        m_i = p[0] + p[1]
        cx = jnp.sum(px_ref[0], axis=0)[:, None]
        cy = jnp.sum(py_ref[0], axis=0)[:, None]
        cz = jnp.sum(pz_ref[0], axis=0)[:, None]
        xo_ref[...] = x_ref[...] + jnp.concatenate([cx, cy, cz], axis=1)
        nh = jax.nn.silu(
            jnp.dot(hb, w1h_ref[...], preferred_element_type=jnp.float32)
            + jnp.dot(m_i, w1m_ref[...], preferred_element_type=jnp.float32)
            + b1_ref[...])
        nh = (jnp.dot(nh, w2_ref[...], preferred_element_type=jnp.float32)
              + b2_ref[...])
        ho_ref[...] = hb + nh

    return pl.pallas_call(
        body,
        grid=(grid,),
        in_specs=[
            pl.BlockSpec((BN, D), lambda i: (i, 0)),
            pl.BlockSpec((BN, 3), lambda i: (i, 0)),
            pl.BlockSpec((NC, BN, D), lambda i: (0, i, 0)),
            pl.BlockSpec((1, NW, BN), lambda i: (i, 0, 0)),
            pl.BlockSpec((1, NW, BN), lambda i: (i, 0, 0)),
            pl.BlockSpec((1, NW, BN), lambda i: (i, 0, 0)),
            pl.BlockSpec((D, D), lambda i: (0, 0)),
            pl.BlockSpec((D, D), lambda i: (0, 0)),
            pl.BlockSpec((1, D), lambda i: (0, 0)),
            pl.BlockSpec((D, D), lambda i: (0, 0)),
            pl.BlockSpec((1, D), lambda i: (0, 0)),
        ],
        out_specs=[
            pl.BlockSpec((BN, D), lambda i: (i, 0)),
            pl.BlockSpec((BN, 3), lambda i: (i, 0)),
        ],
        out_shape=[
            jax.ShapeDtypeStruct((N, D), jnp.float32),
            jax.ShapeDtypeStruct((N, 3), jnp.float32),
        ],
    )(h, x, part, px, py, pz, W1hT, W1mT, bn1, Wn2T, bn2)


def kernel(h, x, edge_index, edge_attr, We1, be1, We2, be2, Wn1, bn1, Wn2,
           bn2, Wc1, bc1, Wc2):
    N, D = h.shape
    src = edge_index[0]
    dst = edge_index[1]
    AT = We1[:, :D].T
    BT = We1[:, D:2 * D].T
    c_row = We1[:, 2 * D].reshape(1, D)
    DmT = We1[:, 2 * D + 1:].T
    xf = x.reshape(-1)
    P, Q = _stage_node_tables(h, x, AT, BT, be1.reshape(1, D))
    zr, rx, ry, rz = _stage_gather(P, Q, xf, src, dst)
    mc, cux, cuy, cuz = _stage_edge_mlp(
        zr, rx, ry, rz, edge_attr, We2.T.astype(jnp.bfloat16),
        be2.reshape(1, D), Wc1.T.astype(jnp.bfloat16), bc1.reshape(1, D),
        Wc2.reshape(1, D), c_row, DmT)
    part, px, py, pz = _stage_scatter(mc, cux, cuy, cuz, dst, N)
    h_out, x_out = _stage_node_mlp(h, x, part, px, py, pz, Wn1[:, :D].T,
                                   Wn1[:, D:].T, bn1.reshape(1, D), Wn2.T,
                                   bn2.reshape(1, D))
    return (h_out, x_out)


# speedup vs baseline: 1.5294x; 1.2842x over previous
import functools

import jax
import jax.numpy as jnp
from jax import lax
from jax.experimental import pallas as pl
from jax.experimental.pallas import tpu as pltpu
from jax.experimental.pallas import tpu_sc as plsc

NC = 2
NS = 16
NW = NC * NS
CH = 80
LANES = 16


def _stage_node_tables(h, x_unused, AT, BT, be1):
    N, D = h.shape
    BN = 1000
    grid = N // BN

    def body(h_ref, at_ref, bt_ref, be1_ref, p_ref, q_ref):
        hb = h_ref[...]
        p_ref[...] = jnp.dot(hb, at_ref[...], preferred_element_type=jnp.float32)
        q_ref[...] = (jnp.dot(hb, bt_ref[...], preferred_element_type=jnp.float32)
                      + be1_ref[...])

    return pl.pallas_call(
        body,
        grid=(grid,),
        in_specs=[
            pl.BlockSpec((BN, D), lambda i: (i, 0)),
            pl.BlockSpec((D, D), lambda i: (0, 0)),
            pl.BlockSpec((D, D), lambda i: (0, 0)),
            pl.BlockSpec((1, D), lambda i: (0, 0)),
        ],
        out_specs=[
            pl.BlockSpec((BN, D), lambda i: (i, 0)),
            pl.BlockSpec((BN, D), lambda i: (i, 0)),
        ],
        out_shape=[jax.ShapeDtypeStruct((N, D), jnp.float32)] * 2,
    )(h, AT, BT, be1)


def _stage_gather(P, Q, xf, src, dst):
    E = src.shape[0]
    N, D = P.shape
    epw = E // NW
    iters = epw // CH
    groups = CH // LANES
    mesh = plsc.VectorSubcoreMesh(core_axis_name="c", subcore_axis_name="s")

    @functools.partial(
        pl.kernel,
        out_type=[
            jax.ShapeDtypeStruct((E, D), jnp.float32),
            jax.ShapeDtypeStruct((E,), jnp.float32),
            jax.ShapeDtypeStruct((E,), jnp.float32),
            jax.ShapeDtypeStruct((E,), jnp.float32),
        ],
        mesh=mesh,
        scratch_types=[
            pltpu.VMEM((3 * N,), jnp.float32),
            pltpu.VMEM((epw,), jnp.int32),
            pltpu.VMEM((epw,), jnp.int32),
            pltpu.VMEM((CH, D), jnp.float32),
            pltpu.VMEM((CH, D), jnp.float32),
            pltpu.VMEM((CH, D), jnp.float32),
            pltpu.VMEM((CH, D), jnp.float32),
            pltpu.VMEM((epw,), jnp.float32),
            pltpu.VMEM((epw,), jnp.float32),
            pltpu.VMEM((epw,), jnp.float32),
            pltpu.SemaphoreType.DMA,
            pltpu.SemaphoreType.DMA,
            pltpu.SemaphoreType.DMA,
            pltpu.SemaphoreType.DMA,
            pltpu.SemaphoreType.DMA,
            pltpu.SemaphoreType.DMA,
        ],
        compiler_params=pltpu.CompilerParams(needs_layout_passes=False),
    )
    def k(p_hbm, q_hbm, xf_hbm, src_hbm, dst_hbm, zr_hbm, rx_hbm, ry_hbm,
          rz_hbm, xtab, idxs, idxd, bufp0, bufq0, bufp1, bufq1, brx, bry,
          brz, semp0, semq0, semp1, semq1, semo0, semo1):
        wid = lax.axis_index("c") * NS + lax.axis_index("s")
        ebase = wid * epw
        pltpu.sync_copy(xf_hbm, xtab)
        pltpu.sync_copy(src_hbm.at[pl.ds(ebase, epw)], idxs)
        pltpu.sync_copy(dst_hbm.at[pl.ds(ebase, epw)], idxd)
        bufp = (bufp0, bufp1)
        bufq = (bufq0, bufq1)
        semp = (semp0, semp1)
        semq = (semq0, semq1)
        semo = (semo0, semo1)

        def fire(kk, st):
            isl = pl.ds(kk * CH, CH)
            pltpu.async_copy(p_hbm.at[idxs.at[isl]], bufp[st], semp[st])
            pltpu.async_copy(q_hbm.at[idxd.at[isl]], bufq[st], semq[st])

        def process(kk, st):
            for g in range(groups):
                sl = pl.ds(kk * CH + g * LANES, LANES)
                s3 = idxs[sl] * 3
                d3 = idxd[sl] * 3
                brx[sl] = (plsc.load_gather(xtab, [s3])
                           - plsc.load_gather(xtab, [d3]))
                bry[sl] = (plsc.load_gather(xtab, [s3 + 1])
                           - plsc.load_gather(xtab, [d3 + 1]))
                brz[sl] = (plsc.load_gather(xtab, [s3 + 2])
                           - plsc.load_gather(xtab, [d3 + 2]))
            pltpu.make_async_copy(p_hbm.at[pl.ds(0, CH)], bufp[st],
                                  semp[st]).wait()
            pltpu.make_async_copy(q_hbm.at[pl.ds(0, CH)], bufq[st],
                                  semq[st]).wait()

            def row(r, carry2):
                for cc in range(D // LANES):
                    sl = pl.ds(cc * LANES, LANES)
                    plsc.addupdate(bufp[st].at[r, sl], bufq[st][r, sl])
                return carry2

            lax.fori_loop(0, CH, row, None)
            pltpu.async_copy(bufp[st], zr_hbm.at[pl.ds(ebase + kk * CH, CH)],
                             semo[st])

        def wait_out(st):
            pltpu.make_async_copy(bufp[st], zr_hbm.at[pl.ds(0, CH)],
                                  semo[st]).wait()

        fire(0, 0)

        def body(j, carry):
            ka = 2 * j + 1

            @pl.when(j > 0)
            def _():
                wait_out(1)

            fire(ka, 1)
            process(ka - 1, 0)
            wait_out(0)
            fire(ka + 1, 0)
            process(ka, 1)
            return carry

        lax.fori_loop(0, (iters - 1) // 2, body, None)
        wait_out(1)
        process(iters - 1, 0)
        wait_out(0)
        pltpu.sync_copy(brx, rx_hbm.at[pl.ds(ebase, epw)])
        pltpu.sync_copy(bry, ry_hbm.at[pl.ds(ebase, epw)])
        pltpu.sync_copy(brz, rz_hbm.at[pl.ds(ebase, epw)])

    return k(P, Q, xf, src, dst)


def _stage_edge_mlp(zr, rx, ry, rz, ea, We2T, be2, Wc1T, bc1, Wc2T, c_row,
                    DmT):
    E, D = zr.shape
    BE = 512
    grid = E // BE

    def body(zr_ref, rx_ref, ry_ref, rz_ref, ea_ref, w2_ref, b2_ref,
             wc1_ref, bc1_ref, wc2_ref, cr_ref, dm_ref, m_ref, cx_ref,
             cy_ref, cz_ref):
        rxb = rx_ref[...]
        ryb = ry_ref[...]
        rzb = rz_ref[...]
        ds2 = (rxb * rxb + ryb * ryb + rzb * rzb)[:, None]
        z = (zr_ref[...] + ds2 * cr_ref[...]
             + jnp.dot(ea_ref[...], dm_ref[...],
                       preferred_element_type=jnp.float32))
        m1 = jax.nn.silu(z).astype(jnp.bfloat16)
        m = jax.nn.silu(
            jnp.dot(m1, w2_ref[...], preferred_element_type=jnp.float32)
            + b2_ref[...])
        cw = jax.nn.silu(
            jnp.dot(m.astype(jnp.bfloat16), wc1_ref[...],
                    preferred_element_type=jnp.float32)
            + bc1_ref[...])
        w0 = jnp.sum(cw * wc2_ref[...], axis=1)
        m_ref[...] = m
        cx_ref[...] = rxb * w0
        cy_ref[...] = ryb * w0
        cz_ref[...] = rzb * w0

    return pl.pallas_call(
        body,
        grid=(grid,),
        in_specs=[
            pl.BlockSpec((BE, D), lambda i: (i, 0)),
            pl.BlockSpec((BE,), lambda i: (i,)),
            pl.BlockSpec((BE,), lambda i: (i,)),
            pl.BlockSpec((BE,), lambda i: (i,)),
            pl.BlockSpec((BE, 4), lambda i: (i, 0)),
            pl.BlockSpec((D, D), lambda i: (0, 0)),
            pl.BlockSpec((1, D), lambda i: (0, 0)),
            pl.BlockSpec((D, D), lambda i: (0, 0)),
            pl.BlockSpec((1, D), lambda i: (0, 0)),
            pl.BlockSpec((1, D), lambda i: (0, 0)),
            pl.BlockSpec((1, D), lambda i: (0, 0)),
            pl.BlockSpec((4, D), lambda i: (0, 0)),
        ],
        out_specs=[
            pl.BlockSpec((BE, D), lambda i: (i, 0)),
            pl.BlockSpec((BE,), lambda i: (i,)),
            pl.BlockSpec((BE,), lambda i: (i,)),
            pl.BlockSpec((BE,), lambda i: (i,)),
        ],
        out_shape=[
            jax.ShapeDtypeStruct((E, D), jnp.float32),
            jax.ShapeDtypeStruct((E,), jnp.float32),
            jax.ShapeDtypeStruct((E,), jnp.float32),
            jax.ShapeDtypeStruct((E,), jnp.float32),
        ],
    )(zr, rx, ry, rz, ea, We2T, be2, Wc1T, bc1, Wc2T, c_row, DmT)


def _stage_scatter_m(mc, dst3, N):
    E, D = mc.shape
    epw = E // NW
    iters = epw // CH
    RPS = 640
    WB = 80
    full_chunks = RPS // WB
    last_chunks = (N - (NS - 1) * RPS) // WB
    mesh = plsc.VectorSubcoreMesh(core_axis_name="c", subcore_axis_name="s")

    @functools.partial(
        pl.kernel,
        out_type=jax.ShapeDtypeStruct((NC, N, D), jnp.float32),
        mesh=mesh,
        scratch_types=[
            pltpu.VMEM((iters, CH), jnp.int32),
            pltpu.VMEM((CH, D), jnp.float32),
            pltpu.VMEM((CH, D), jnp.float32),
            pltpu.VMEM_SHARED((N, D), jnp.float32),
            pltpu.SemaphoreType.DMA,
            pltpu.SemaphoreType.DMA,
        ],
        compiler_params=pltpu.CompilerParams(needs_layout_passes=False),
    )
    def k(mc_hbm, dst_hbm, part_hbm, idx2, mb0, mb1, acc, sl0, sl1):
        c = lax.axis_index("c")
        s = lax.axis_index("s")
        wid = c * NS + s
        nchunks = jnp.where(s < NS - 1, full_chunks, last_chunks)
        zv = jnp.zeros((LANES,), jnp.float32)
        mb = (mb0, mb1)
        sem = (sl0, sl1)
        pltpu.sync_copy(dst_hbm.at[wid], idx2)

        def zrow(r, carry):
            for cc in range(D // LANES):
                mb0[r, pl.ds(cc * LANES, LANES)] = zv
            return carry

        lax.fori_loop(0, WB, zrow, None)

        def zacc(j, carry):
            pltpu.sync_copy(mb0, acc.at[pl.ds(s * RPS + j * WB, WB)])
            return carry

        lax.fori_loop(0, nchunks, zacc, None)
        plsc.subcore_barrier()

        def fire(kk, st):
            pltpu.async_copy(mc_hbm.at[pl.ds(wid * epw + kk * CH, CH)],
                             mb[st], sem[st])

        def wait_in(st):
            pltpu.make_async_copy(mc_hbm.at[pl.ds(0, CH)], mb[st],
                                  sem[st]).wait()

        def scat(kk, st):
            pltpu.sync_copy(mb[st], acc.at[idx2.at[kk]], add=True)

        fire(0, 0)

        def body(j, carry):
            ka = 2 * j + 1
            fire(ka, 1)
            wait_in(0)
            scat(ka - 1, 0)
            fire(ka + 1, 0)
            wait_in(1)
            scat(ka, 1)
            return carry

        lax.fori_loop(0, (iters - 1) // 2, body, None)
        wait_in(0)
        scat(iters - 1, 0)
        plsc.subcore_barrier()

        def wb(j, carry):
            row0 = s * RPS + j * WB
            pltpu.sync_copy(acc.at[pl.ds(row0, WB)], mb0)
            pltpu.sync_copy(mb0, part_hbm.at[c, pl.ds(row0, WB)])
            return carry

        lax.fori_loop(0, nchunks, wb, None)

    return k(mc, dst3)


def _stage_scatter_xyz(cux, cuy, cuz, dst, N):
    E = cux.shape[0]
    epw = E // NW
    groups = epw // LANES
    mesh = plsc.VectorSubcoreMesh(core_axis_name="c", subcore_axis_name="s")

    @functools.partial(
        pl.kernel,
        out_type=[
            jax.ShapeDtypeStruct((NW * N,), jnp.float32),
            jax.ShapeDtypeStruct((NW * N,), jnp.float32),
            jax.ShapeDtypeStruct((NW * N,), jnp.float32),
        ],
        mesh=mesh,
        scratch_types=[
            pltpu.VMEM((epw,), jnp.int32),
            pltpu.VMEM((epw,), jnp.float32),
            pltpu.VMEM((epw,), jnp.float32),
            pltpu.VMEM((epw,), jnp.float32),
            pltpu.VMEM((N,), jnp.float32),
            pltpu.VMEM((N,), jnp.float32),
            pltpu.VMEM((N,), jnp.float32),
        ],
        compiler_params=pltpu.CompilerParams(needs_layout_passes=False),
    )
    def k(cux_hbm, cuy_hbm, cuz_hbm, dst_hbm, px_hbm, py_hbm, pz_hbm,
          idxd, bx, by, bz, ax, ay, az):
        c = lax.axis_index("c")
        s = lax.axis_index("s")
        wid = c * NS + s
        ebase = wid * epw
        pltpu.sync_copy(dst_hbm.at[pl.ds(ebase, epw)], idxd)
        pltpu.sync_copy(cux_hbm.at[pl.ds(ebase, epw)], bx)
        pltpu.sync_copy(cuy_hbm.at[pl.ds(ebase, epw)], by)
        pltpu.sync_copy(cuz_hbm.at[pl.ds(ebase, epw)], bz)
        zv = jnp.zeros((LANES,), jnp.float32)

        def zcoord(r, carry):
            sl = pl.ds(r * LANES, LANES)
            ax[sl] = zv
            ay[sl] = zv
            az[sl] = zv
            return carry

        lax.fori_loop(0, N // LANES, zcoord, None)

        def body(g, carry):
            sl = pl.ds(g * LANES, LANES)
            dv = idxd[sl]
            plsc.addupdate_scatter(ax, [dv], bx[sl])
            plsc.addupdate_scatter(ay, [dv], by[sl])
            plsc.addupdate_scatter(az, [dv], bz[sl])
            return carry

        lax.fori_loop(0, groups, body, None)
        pltpu.sync_copy(ax, px_hbm.at[pl.ds(wid * N, N)])
        pltpu.sync_copy(ay, py_hbm.at[pl.ds(wid * N, N)])
        pltpu.sync_copy(az, pz_hbm.at[pl.ds(wid * N, N)])

    return k(cux, cuy, cuz, dst)


def _stage_node_mlp(h, x, part, px, py, pz, W1hT, W1mT, bn1, Wn2T, bn2):
    N, D = h.shape
    BN = 1000
    grid = N // BN

    def body(h_ref, x_ref, part_ref, px_ref, py_ref, pz_ref, w1h_ref,
             w1m_ref, b1_ref, w2_ref, b2_ref, ho_ref, xo_ref):
        hb = h_ref[...]
        p = part_ref[...]
        m_i = p[0] + p[1]
        cx = jnp.sum(px_ref[0], axis=0)[:, None]
        cy = jnp.sum(py_ref[0], axis=0)[:, None]
        cz = jnp.sum(pz_ref[0], axis=0)[:, None]
        xo_ref[...] = x_ref[...] + jnp.concatenate([cx, cy, cz], axis=1)
        nh = jax.nn.silu(
            jnp.dot(hb, w1h_ref[...], preferred_element_type=jnp.float32)
            + jnp.dot(m_i, w1m_ref[...], preferred_element_type=jnp.float32)
            + b1_ref[...])
        nh = (jnp.dot(nh, w2_ref[...], preferred_element_type=jnp.float32)
              + b2_ref[...])
        ho_ref[...] = hb + nh

    return pl.pallas_call(
        body,
        grid=(grid,),
        in_specs=[
            pl.BlockSpec((BN, D), lambda i: (i, 0)),
            pl.BlockSpec((BN, 3), lambda i: (i, 0)),
            pl.BlockSpec((NC, BN, D), lambda i: (0, i, 0)),
            pl.BlockSpec((1, NW, BN), lambda i: (i, 0, 0)),
            pl.BlockSpec((1, NW, BN), lambda i: (i, 0, 0)),
            pl.BlockSpec((1, NW, BN), lambda i: (i, 0, 0)),
            pl.BlockSpec((D, D), lambda i: (0, 0)),
            pl.BlockSpec((D, D), lambda i: (0, 0)),
            pl.BlockSpec((1, D), lambda i: (0, 0)),
            pl.BlockSpec((D, D), lambda i: (0, 0)),
            pl.BlockSpec((1, D), lambda i: (0, 0)),
        ],
        out_specs=[
            pl.BlockSpec((BN, D), lambda i: (i, 0)),
            pl.BlockSpec((BN, 3), lambda i: (i, 0)),
        ],
        out_shape=[
            jax.ShapeDtypeStruct((N, D), jnp.float32),
            jax.ShapeDtypeStruct((N, 3), jnp.float32),
        ],
    )(h, x, part, px, py, pz, W1hT, W1mT, bn1, Wn2T, bn2)


def kernel(h, x, edge_index, edge_attr, We1, be1, We2, be2, Wn1, bn1, Wn2,
           bn2, Wc1, bc1, Wc2):
    N, D = h.shape
    E = edge_index.shape[1]
    src = edge_index[0]
    dst = edge_index[1]
    AT = We1[:, :D].T
    BT = We1[:, D:2 * D].T
    c_row = We1[:, 2 * D].reshape(1, D)
    DmT = We1[:, 2 * D + 1:].T
    xf = x.reshape(-1)
    P, Q = _stage_node_tables(h, x, AT, BT, be1.reshape(1, D))
    zr, rx, ry, rz = _stage_gather(P, Q, xf, src, dst)
    mc, cux, cuy, cuz = _stage_edge_mlp(
        zr, rx, ry, rz, edge_attr, We2.T.astype(jnp.bfloat16),
        be2.reshape(1, D), Wc1.T.astype(jnp.bfloat16), bc1.reshape(1, D),
        Wc2.reshape(1, D), c_row, DmT)
    dst3 = dst.reshape(NW, E // (NW * CH), CH)
    part = _stage_scatter_m(mc, dst3, N)
    px, py, pz = _stage_scatter_xyz(cux, cuy, cuz, dst, N)
    BN = 1000

    def t(a):
        return a.reshape(NW, N // BN, BN).transpose(1, 0, 2)

    px, py, pz = t(px), t(py), t(pz)
    h_out, x_out = _stage_node_mlp(h, x, part, px, py, pz, Wn1[:, :D].T,
                                   Wn1[:, D:].T, bn1.reshape(1, D), Wn2.T,
                                   bn2.reshape(1, D))
    return (h_out, x_out)
